# baseline (speedup 1.0000x reference)
import jax
import jax.numpy as jnp
from jax import lax
from jax.experimental import pallas as pl
from jax.experimental.pallas import tpu as pltpu
from jax.experimental.pallas import tpu_sc as plsc

N_CORES = 2
N_SUBCORES = 16
N_WORKERS = N_CORES * N_SUBCORES
LANES = 16

N_RELATION = 1000
BATCH = 16384
DIM = 128
B_W = BATCH // N_WORKERS
CHUNK = 128
N_CHUNKS = B_W // CHUNK
GROUPS = CHUNK // LANES
SLICES = DIM // LANES
TPAD = 24


def _sc_body(h_hbm, r_hbm, t_hbm, ent_hbm, rel_hbm, out_hbm,
             h_idx, r_idx, t_idx,
             hb0, rb0, tb0, hb1, rb1, tb1,
             out_v, tr, sem0, sem1, sem2):
    sid = lax.axis_index("s")
    wid = sid * N_CORES + lax.axis_index("c")
    base = pl.multiple_of(wid * B_W, B_W)

    bufs = ((hb0, rb0, tb0, sem0), (hb1, rb1, tb1, sem1))

    def start(c):
        hb, rb, tb, sem = bufs[c % 2]
        lo = c * CHUNK
        return (
            pltpu.async_copy(ent_hbm.at[h_idx.at[pl.ds(lo, CHUNK)]], hb, sem),
            pltpu.async_copy(rel_hbm.at[r_idx.at[pl.ds(lo, CHUNK)]], rb, sem),
            pltpu.async_copy(ent_hbm.at[t_idx.at[pl.ds(lo, CHUNK)]], tb, sem),
        )

    hb_0, rb_0, tb_0, gsem0 = bufs[0]
    isem = sem1
    di0 = pltpu.async_copy(h_hbm.at[pl.ds(base, B_W)], h_idx, isem)
    di1 = pltpu.async_copy(r_hbm.at[pl.ds(base, B_W)], r_idx, isem)
    di2 = pltpu.async_copy(t_hbm.at[pl.ds(base, B_W)], t_idx, isem)
    di0.wait()
    g0 = pltpu.async_copy(ent_hbm.at[h_idx.at[pl.ds(0, CHUNK)]], hb_0, gsem0)
    di1.wait()
    g1 = pltpu.async_copy(rel_hbm.at[r_idx.at[pl.ds(0, CHUNK)]], rb_0, gsem0)
    di2.wait()
    g2 = pltpu.async_copy(ent_hbm.at[t_idx.at[pl.ds(0, CHUNK)]], tb_0, gsem0)

    iota = lax.iota(jnp.int32, LANES)

    def compute(c):
        hb, rb, tb, _ = bufs[c % 2]

        def group(g, carry):
            row0 = pl.multiple_of(g * LANES, LANES)

            def one_row(j, carry2):
                row = row0 + j
                acc = jnp.zeros((LANES,), jnp.float32)
                for s in range(SLICES):
                    sl = pl.ds(s * LANES, LANES)
                    acc = acc + jnp.abs(hb[row, sl] + rb[row, sl] - tb[row, sl])
                tr[pl.ds(j * TPAD, LANES)] = acc
                return carry2

            lax.fori_loop(0, LANES, one_row, 0)
            tot = plsc.load_gather(tr, [iota * TPAD])
            for i in range(1, LANES):
                tot = tot + plsc.load_gather(tr, [iota * TPAD + i])
            out_v[pl.ds(c * CHUNK + row0, LANES)] = -tot
            return carry

        lax.fori_loop(0, GROUPS, group, 0)

    descs = [(g0, g1, g2)]
    osem = sem2
    odescs = []
    for c in range(N_CHUNKS):
        if c + 1 < N_CHUNKS:
            descs.append(start(c + 1))
        for d in descs[c]:
            d.wait()
        compute(c)
        odescs.append(pltpu.async_copy(
            out_v.at[pl.ds(c * CHUNK, CHUNK)],
            out_hbm.at[pl.ds(base + c * CHUNK, CHUNK)], osem))
    for d in odescs:
        d.wait()


def _make_kernel():
    mesh = plsc.VectorSubcoreMesh(core_axis_name="c", subcore_axis_name="s",
                                  num_cores=N_CORES, num_subcores=N_SUBCORES)
    return pl.kernel(
        _sc_body,
        out_type=jax.ShapeDtypeStruct((BATCH,), jnp.float32),
        mesh=mesh,
        compiler_params=pltpu.CompilerParams(
            needs_layout_passes=False,
            skip_device_barrier=True,
            disable_bounds_checks=True,
            disable_semaphore_checks=True,
        ),
        scratch_types=(
            [pltpu.VMEM((B_W,), jnp.int32)] * 3
            + [pltpu.VMEM((CHUNK, DIM), jnp.float32)] * 6
            + [pltpu.VMEM((B_W,), jnp.float32),
               pltpu.VMEM((LANES * TPAD,), jnp.float32)]
            + [pltpu.SemaphoreType.DMA] * 3
        ),
    )


@jax.jit
def kernel(h, r, t, entity_embedding, relation_embedding):
    fn = _make_kernel()
    return fn(h.astype(jnp.int32), r.astype(jnp.int32), t.astype(jnp.int32),
              entity_embedding, relation_embedding)


# speedup vs baseline: 1.0511x; 1.0511x over previous
import jax
import jax.numpy as jnp
from jax import lax
from jax.experimental import pallas as pl
from jax.experimental.pallas import tpu as pltpu
from jax.experimental.pallas import tpu_sc as plsc

N_CORES = 2
N_SUBCORES = 16
N_WORKERS = N_CORES * N_SUBCORES
LANES = 16

N_RELATION = 1000
BATCH = 16384
DIM = 128
B_W = BATCH // N_WORKERS
CHUNK = 128
SIZES = (32, 96, 128, 128, 96, 32)
OFFS = (0, 32, 128, 256, 384, 480)
N_CHUNKS = len(SIZES)
SLICES = DIM // LANES
TPAD = 24


def _sc_body(h_hbm, r_hbm, t_hbm, ent_hbm, rel_hbm, out_hbm,
             h_idx, r_idx, t_idx,
             hb0, rb0, tb0, hb1, rb1, tb1,
             out_v, tr, sem0, sem1, sem2):
    sid = lax.axis_index("s")
    wid = sid * N_CORES + lax.axis_index("c")
    base = pl.multiple_of(wid * B_W, B_W)

    bufs = ((hb0, rb0, tb0, sem0), (hb1, rb1, tb1, sem1))

    def start(c):
        hb, rb, tb, sem = bufs[c % 2]
        lo, n = OFFS[c], SIZES[c]
        return (
            pltpu.async_copy(ent_hbm.at[h_idx.at[pl.ds(lo, n)]],
                             hb.at[pl.ds(0, n)], sem),
            pltpu.async_copy(rel_hbm.at[r_idx.at[pl.ds(lo, n)]],
                             rb.at[pl.ds(0, n)], sem),
            pltpu.async_copy(ent_hbm.at[t_idx.at[pl.ds(lo, n)]],
                             tb.at[pl.ds(0, n)], sem),
        )

    hb_0, rb_0, tb_0, gsem0 = bufs[0]
    n0 = SIZES[0]
    isem = sem1
    di0 = pltpu.async_copy(h_hbm.at[pl.ds(base, B_W)], h_idx, isem)
    di1 = pltpu.async_copy(r_hbm.at[pl.ds(base, B_W)], r_idx, isem)
    di2 = pltpu.async_copy(t_hbm.at[pl.ds(base, B_W)], t_idx, isem)
    di0.wait()
    g0 = pltpu.async_copy(ent_hbm.at[h_idx.at[pl.ds(0, n0)]],
                          hb_0.at[pl.ds(0, n0)], gsem0)
    di1.wait()
    g1 = pltpu.async_copy(rel_hbm.at[r_idx.at[pl.ds(0, n0)]],
                          rb_0.at[pl.ds(0, n0)], gsem0)
    di2.wait()
    g2 = pltpu.async_copy(ent_hbm.at[t_idx.at[pl.ds(0, n0)]],
                          tb_0.at[pl.ds(0, n0)], gsem0)

    iota = lax.iota(jnp.int32, LANES)

    def compute(c):
        hb, rb, tb, _ = bufs[c % 2]
        lo = OFFS[c]

        def group(g, carry):
            row0 = pl.multiple_of(g * LANES, LANES)

            def one_row(j, carry2):
                row = row0 + j
                acc = jnp.zeros((LANES,), jnp.float32)
                for s in range(SLICES):
                    sl = pl.ds(s * LANES, LANES)
                    acc = acc + jnp.abs(hb[row, sl] + rb[row, sl] - tb[row, sl])
                tr[pl.ds(j * TPAD, LANES)] = acc
                return carry2

            lax.fori_loop(0, LANES, one_row, 0)
            tot = plsc.load_gather(tr, [iota * TPAD])
            for i in range(1, LANES):
                tot = tot + plsc.load_gather(tr, [iota * TPAD + i])
            out_v[pl.ds(lo + row0, LANES)] = -tot
            return carry

        lax.fori_loop(0, SIZES[c] // LANES, group, 0)

    descs = [(g0, g1, g2)]
    osem = sem2
    odescs = []
    for c in range(N_CHUNKS):
        if c + 1 < N_CHUNKS:
            descs.append(start(c + 1))
        for d in descs[c]:
            d.wait()
        compute(c)
        odescs.append(pltpu.async_copy(
            out_v.at[pl.ds(OFFS[c], SIZES[c])],
            out_hbm.at[pl.ds(base + OFFS[c], SIZES[c])], osem))
    for d in odescs:
        d.wait()


def _make_kernel():
    mesh = plsc.VectorSubcoreMesh(core_axis_name="c", subcore_axis_name="s",
                                  num_cores=N_CORES, num_subcores=N_SUBCORES)
    return pl.kernel(
        _sc_body,
        out_type=jax.ShapeDtypeStruct((BATCH,), jnp.float32),
        mesh=mesh,
        compiler_params=pltpu.CompilerParams(needs_layout_passes=False),
        scratch_types=(
            [pltpu.VMEM((B_W,), jnp.int32)] * 3
            + [pltpu.VMEM((CHUNK, DIM), jnp.float32)] * 6
            + [pltpu.VMEM((B_W,), jnp.float32),
               pltpu.VMEM((LANES * TPAD,), jnp.float32)]
            + [pltpu.SemaphoreType.DMA] * 3
        ),
    )


@jax.jit
def kernel(h, r, t, entity_embedding, relation_embedding):
    fn = _make_kernel()
    return fn(h.astype(jnp.int32), r.astype(jnp.int32), t.astype(jnp.int32),
              entity_embedding, relation_embedding)


# speedup vs baseline: 1.0521x; 1.0010x over previous
import jax
import jax.numpy as jnp
from jax import lax
from jax.experimental import pallas as pl
from jax.experimental.pallas import tpu as pltpu
from jax.experimental.pallas import tpu_sc as plsc

N_CORES = 2
N_SUBCORES = 16
N_WORKERS = N_CORES * N_SUBCORES
LANES = 16

BATCH = 16384
DIM = 128
B_W = BATCH // N_WORKERS
CHUNK = 128
SIZES = (32, 96, 128, 128, 96, 32)
OFFS = (0, 32, 128, 256, 384, 480)
N_CHUNKS = len(SIZES)
SLICES = DIM // LANES
TPAD = 24


def _sc_body(h_hbm, r_hbm, t_hbm, ent_hbm, rel_hbm, out_hbm,
             h_idx, r_idx, t_idx,
             hb0, rb0, tb0, hb1, rb1, tb1,
             out_v, tr, sem0, sem1, sem2):
    sid = lax.axis_index("s")
    wid = sid * N_CORES + lax.axis_index("c")
    base = pl.multiple_of(wid * B_W, B_W)

    bufs = ((hb0, rb0, tb0, sem0), (hb1, rb1, tb1, sem1))

    def start(c):
        hb, rb, tb, sem = bufs[c % 2]
        lo, n = OFFS[c], SIZES[c]
        return (
            pltpu.async_copy(ent_hbm.at[h_idx.at[pl.ds(lo, n)]],
                             hb.at[pl.ds(0, n)], sem),
            pltpu.async_copy(rel_hbm.at[r_idx.at[pl.ds(lo, n)]],
                             rb.at[pl.ds(0, n)], sem),
            pltpu.async_copy(ent_hbm.at[t_idx.at[pl.ds(lo, n)]],
                             tb.at[pl.ds(0, n)], sem),
        )

    hb_0, rb_0, tb_0, gsem0 = bufs[0]
    n0 = SIZES[0]
    isem = sem1
    di0 = pltpu.async_copy(h_hbm.at[pl.ds(base, B_W)], h_idx, isem)
    di1 = pltpu.async_copy(r_hbm.at[pl.ds(base, B_W)], r_idx, isem)
    di2 = pltpu.async_copy(t_hbm.at[pl.ds(base, B_W)], t_idx, isem)
    di0.wait()
    g0 = pltpu.async_copy(ent_hbm.at[h_idx.at[pl.ds(0, n0)]],
                          hb_0.at[pl.ds(0, n0)], gsem0)
    di1.wait()
    g1 = pltpu.async_copy(rel_hbm.at[r_idx.at[pl.ds(0, n0)]],
                          rb_0.at[pl.ds(0, n0)], gsem0)
    di2.wait()
    g2 = pltpu.async_copy(ent_hbm.at[t_idx.at[pl.ds(0, n0)]],
                          tb_0.at[pl.ds(0, n0)], gsem0)

    iota = lax.iota(jnp.int32, LANES)

    def compute(c):
        hb, rb, tb, _ = bufs[c % 2]
        lo = OFFS[c]

        def group(g, carry):
            row0 = pl.multiple_of(g * LANES, LANES)

            def one_row(j, carry2):
                row = row0 + j
                acc = jnp.zeros((LANES,), jnp.float32)
                for s in range(SLICES):
                    sl = pl.ds(s * LANES, LANES)
                    acc = acc + jnp.abs(hb[row, sl] + rb[row, sl] - tb[row, sl])
                tr[pl.ds(j * TPAD, LANES)] = acc
                return carry2

            lax.fori_loop(0, LANES, one_row, 0)
            tot = plsc.load_gather(tr, [iota * TPAD])
            for i in range(1, LANES):
                tot = tot + plsc.load_gather(tr, [iota * TPAD + i])
            out_v[pl.ds(lo + row0, LANES)] = -tot
            return carry

        lax.fori_loop(0, SIZES[c] // LANES, group, 0)

    descs = [(g0, g1, g2)]
    osem = sem2
    odescs = []
    for c in range(N_CHUNKS):
        if c + 1 < N_CHUNKS:
            descs.append(start(c + 1))
        for d in descs[c]:
            d.wait()
        compute(c)
        odescs.append(pltpu.async_copy(
            out_v.at[pl.ds(OFFS[c], SIZES[c])],
            out_hbm.at[pl.ds(base + OFFS[c], SIZES[c])], osem))
    for d in odescs:
        d.wait()


def _make_kernel():
    mesh = plsc.VectorSubcoreMesh(core_axis_name="c", subcore_axis_name="s",
                                  num_cores=N_CORES, num_subcores=N_SUBCORES)
    return pl.kernel(
        _sc_body,
        out_type=jax.ShapeDtypeStruct((BATCH,), jnp.float32),
        mesh=mesh,
        compiler_params=pltpu.CompilerParams(needs_layout_passes=False),
        scratch_types=(
            [pltpu.VMEM((B_W,), jnp.int32)] * 3
            + [pltpu.VMEM((CHUNK, DIM), jnp.float32)] * 6
            + [pltpu.VMEM((B_W,), jnp.float32),
               pltpu.VMEM((LANES * TPAD,), jnp.float32)]
            + [pltpu.SemaphoreType.DMA] * 3
        ),
    )


@jax.jit
def kernel(h, r, t, entity_embedding, relation_embedding):
    fn = _make_kernel()
    return fn(h.astype(jnp.int32), r.astype(jnp.int32), t.astype(jnp.int32),
              entity_embedding, relation_embedding)
